# baseline (speedup 1.0000x reference)
import jax
import jax.numpy as jnp
from jax import lax
from jax.experimental import pallas as pl
from jax.experimental.pallas import tpu as pltpu
from jax.experimental.pallas import tpu_sc as plsc

EMBED_DIM = 32
BATCH = 16384
NUM_CORES = 2
NUM_SUBCORES = 16
NUM_WORKERS = NUM_CORES * NUM_SUBCORES
ROWS_PER_WORKER = BATCH // NUM_WORKERS
IDX_CHUNK = 128
CHUNKS = ROWS_PER_WORKER // IDX_CHUNK
LANES = 16


def _gmf_body(user_ids_ref, item_ids_ref, user_table_ref, item_table_ref,
              out_ref, uidx_v, iidx_v, urows_v, irows_v, out_v, sem_u, sem_i):
    wid = lax.axis_index("s") * NUM_CORES + lax.axis_index("c")
    base = wid * ROWS_PER_WORKER

    pltpu.sync_copy(user_ids_ref.at[pl.ds(wid * CHUNKS, CHUNKS)], uidx_v)
    pltpu.sync_copy(item_ids_ref.at[pl.ds(wid * CHUNKS, CHUNKS)], iidx_v)

    copies = []
    for j in range(CHUNKS):
        sl = pl.ds(j * IDX_CHUNK, IDX_CHUNK)
        copies.append(pltpu.async_copy(
            user_table_ref.at[uidx_v.at[j]], urows_v.at[sl], sem_u))
        copies.append(pltpu.async_copy(
            item_table_ref.at[iidx_v.at[j]], irows_v.at[sl], sem_i))
    for c in copies:
        c.wait()

    lane = lax.iota(jnp.int32, LANES)
    first_lane = lane == 0
    half = EMBED_DIM // 2
    perms = [lane ^ (1 << k) for k in range(4)]

    def shuffle(x, perm):
        return lax.gather(
            x, perm[:, None],
            lax.GatherDimensionNumbers(
                offset_dims=(), collapsed_slice_dims=(0,),
                start_index_map=(0,)),
            slice_sizes=(1,),
            mode=lax.GatherScatterMode.PROMISE_IN_BOUNDS)

    def group_body(g, _):
        acc = jnp.zeros((LANES,), jnp.float32)
        for k in range(LANES):
            r = g * LANES + k
            u0 = urows_v[r, pl.ds(0, LANES)]
            u1 = urows_v[r, pl.ds(half, LANES)]
            i0 = irows_v[r, pl.ds(0, LANES)]
            i1 = irows_v[r, pl.ds(half, LANES)]
            s = u0 * i0 + u1 * i1
            for perm in perms:
                s = s + shuffle(s, perm)
            acc = jnp.where(lane == k, s, acc)
        out_v[pl.ds(g * LANES, LANES)] = 5.0 / (1.0 + jnp.exp(-acc))
        return 0

    lax.fori_loop(0, ROWS_PER_WORKER // LANES, group_body, 0)

    pltpu.sync_copy(out_v, out_ref.at[pl.ds(base, ROWS_PER_WORKER)])


def kernel(user_ids, item_ids, user_table, item_table):
    uids = user_ids.astype(jnp.int32).reshape(NUM_WORKERS * CHUNKS, IDX_CHUNK)
    iids = item_ids.astype(jnp.int32).reshape(NUM_WORKERS * CHUNKS, IDX_CHUNK)

    mesh = plsc.VectorSubcoreMesh(core_axis_name="c", subcore_axis_name="s")
    f = pl.kernel(
        _gmf_body,
        out_type=jax.ShapeDtypeStruct((BATCH,), jnp.float32),
        mesh=mesh,
        scratch_types=[
            pltpu.VMEM((CHUNKS, IDX_CHUNK), jnp.int32),
            pltpu.VMEM((CHUNKS, IDX_CHUNK), jnp.int32),
            pltpu.VMEM((ROWS_PER_WORKER, EMBED_DIM), jnp.float32),
            pltpu.VMEM((ROWS_PER_WORKER, EMBED_DIM), jnp.float32),
            pltpu.VMEM((ROWS_PER_WORKER,), jnp.float32),
            pltpu.SemaphoreType.DMA,
            pltpu.SemaphoreType.DMA,
        ],
        compiler_params=pltpu.CompilerParams(use_tc_tiling_on_sc=False),
    )
    return f(uids, iids, user_table, item_table)
